# Initial kernel scaffold; baseline (speedup 1.0000x reference)
import functools

import jax
import jax.numpy as jnp
from jax import lax
from jax.experimental import pallas as pl
from jax.experimental.pallas import tpu as pltpu
from jax.experimental.pallas import tpu_sc as plsc

N = 10000
D = 128
E = 320000
CHUNK = 128
NCH = E // CHUNK
NC = 2
NS = 16
CH_PER_SC = NCH // NC
CH_PER_TILE = CH_PER_SC // NS
CH_LEFT = CH_PER_SC - CH_PER_TILE * NS
ROWS_PER_TILE = N // NS

_MESH = plsc.VectorSubcoreMesh(core_axis_name="c", subcore_axis_name="s")


def _fill_zero(ref, nrows, ncols):
    z = jnp.zeros((16,), jnp.float32)

    def body(i, _):
        for k in range(ncols // 16):
            ref[i, pl.ds(k * 16, 16)] = z
        return 0

    lax.fori_loop(0, nrows, body, 0)


@functools.partial(
    pl.kernel,
    out_type=jax.ShapeDtypeStruct((2 * N, 16), jnp.float32),
    mesh=_MESH,
    scratch_types=[
        pltpu.VMEM((CHUNK, 16), jnp.float32),
        pltpu.VMEM((CHUNK,), jnp.int32),
        pltpu.VMEM((ROWS_PER_TILE, 16), jnp.float32),
        pltpu.VMEM_SHARED((N, 16), jnp.float32),
    ],
)
def _deg_kernel(dst_hbm, out_hbm, ones_v, dst_v, zero_v, acc_sh):
    c = lax.axis_index("c")
    s = lax.axis_index("s")

    one = jnp.ones((16,), jnp.float32)

    def fill_ones(i, _):
        ones_v[i, :] = one
        return 0

    lax.fori_loop(0, CHUNK, fill_ones, 0)
    _fill_zero(zero_v, ROWS_PER_TILE, 16)

    pltpu.sync_copy(zero_v, acc_sh.at[pl.ds(s * ROWS_PER_TILE, ROWS_PER_TILE)])
    plsc.subcore_barrier()

    base = c * CH_PER_SC + s * CH_PER_TILE

    def body(j, _):
        pltpu.sync_copy(dst_hbm.at[base + j], dst_v)
        pltpu.sync_copy(ones_v, acc_sh.at[dst_v], add=True)
        return 0

    lax.fori_loop(0, CH_PER_TILE, body, 0)

    @pl.when(s < CH_LEFT)
    def _():
        ch = c * CH_PER_SC + NS * CH_PER_TILE + s
        pltpu.sync_copy(dst_hbm.at[ch], dst_v)
        pltpu.sync_copy(ones_v, acc_sh.at[dst_v], add=True)

    plsc.subcore_barrier()
    r0 = s * ROWS_PER_TILE
    pltpu.sync_copy(
        acc_sh.at[pl.ds(r0, ROWS_PER_TILE)],
        out_hbm.at[pl.ds(c * N + r0, ROWS_PER_TILE)],
    )


ZROWS = 125


@functools.partial(
    pl.kernel,
    out_type=jax.ShapeDtypeStruct((2 * N, D), jnp.float32),
    mesh=_MESH,
    scratch_types=[
        pltpu.VMEM((CHUNK, D), jnp.float32),
        pltpu.VMEM((CHUNK,), jnp.int32),
        pltpu.VMEM((CHUNK,), jnp.int32),
        pltpu.VMEM((ZROWS, D), jnp.float32),
        pltpu.VMEM_SHARED((N, D), jnp.float32),
        pltpu.SemaphoreType.DMA,
    ],
)
def _scatter_kernel(xs_hbm, src_hbm, dst_hbm, out_hbm,
                    rows_v, src_v, dst_v, zero_v, acc_sh, sem):
    c = lax.axis_index("c")
    s = lax.axis_index("s")

    _fill_zero(zero_v, ZROWS, D)
    for k in range(ROWS_PER_TILE // ZROWS):
        pltpu.sync_copy(
            zero_v, acc_sh.at[pl.ds(s * ROWS_PER_TILE + k * ZROWS, ZROWS)])
    plsc.subcore_barrier()

    base = c * CH_PER_SC + s * CH_PER_TILE

    def do_chunk(ch):
        pltpu.sync_copy(src_hbm.at[ch], src_v)
        pltpu.sync_copy(dst_hbm.at[ch], dst_v)
        pltpu.async_copy(xs_hbm.at[src_v], rows_v, sem).wait()
        pltpu.sync_copy(rows_v, acc_sh.at[dst_v], add=True)

    def body(j, _):
        do_chunk(base + j)
        return 0

    lax.fori_loop(0, CH_PER_TILE, body, 0)

    @pl.when(s < CH_LEFT)
    def _():
        do_chunk(c * CH_PER_SC + NS * CH_PER_TILE + s)

    plsc.subcore_barrier()
    r0 = s * ROWS_PER_TILE
    pltpu.sync_copy(
        acc_sh.at[pl.ds(r0, ROWS_PER_TILE)],
        out_hbm.at[pl.ds(c * N + r0, ROWS_PER_TILE)],
    )


MBLK = 1000
GRID = N // MBLK


def _dis(d0_ref, d1_ref):
    deg = d0_ref[:, 0:1] + d1_ref[:, 0:1] + 1.0
    return lax.rsqrt(deg)


def _mm_scale_body(x_ref, w_ref, d0_ref, d1_ref, o_ref):
    dis = _dis(d0_ref, d1_ref)
    xw = jnp.dot(x_ref[...], w_ref[...], preferred_element_type=jnp.float32)
    o_ref[...] = xw * dis


def _combine_mm_body(a0_ref, a1_ref, xs_ref, d0_ref, d1_ref, b_ref, w_ref,
                     o_ref):
    dis = _dis(d0_ref, d1_ref)
    tot = (a0_ref[...] + a1_ref[...] + xs_ref[...]) * dis + b_ref[...]
    h = jnp.maximum(tot, 0.0)
    o_ref[...] = jnp.dot(h, w_ref[...], preferred_element_type=jnp.float32) * dis


def _final_body(a0_ref, a1_ref, xs_ref, d0_ref, d1_ref, b_ref, wl_ref, bl_ref,
                o_ref):
    dis = _dis(d0_ref, d1_ref)
    tot = (a0_ref[...] + a1_ref[...] + xs_ref[...]) * dis + b_ref[...]
    h = jnp.maximum(tot, 0.0)
    o_ref[...] = (
        jnp.dot(h, wl_ref[...], preferred_element_type=jnp.float32) + bl_ref[...]
    )


def _row_spec(cols):
    return pl.BlockSpec((MBLK, cols), lambda i: (i, 0))


def _full_spec(r, cols):
    return pl.BlockSpec((r, cols), lambda i: (0, 0))


_mm_scale = pl.pallas_call(
    _mm_scale_body,
    grid=(GRID,),
    in_specs=[_row_spec(D), _full_spec(D, D), _row_spec(16), _row_spec(16)],
    out_specs=_row_spec(D),
    out_shape=jax.ShapeDtypeStruct((N, D), jnp.float32),
)

_combine_mm = pl.pallas_call(
    _combine_mm_body,
    grid=(GRID,),
    in_specs=[_row_spec(D), _row_spec(D), _row_spec(D), _row_spec(16),
              _row_spec(16), _full_spec(1, D), _full_spec(D, D)],
    out_specs=_row_spec(D),
    out_shape=jax.ShapeDtypeStruct((N, D), jnp.float32),
)

_final_mm = pl.pallas_call(
    _final_body,
    grid=(GRID,),
    in_specs=[_row_spec(D), _row_spec(D), _row_spec(D), _row_spec(16),
              _row_spec(16), _full_spec(1, D), _full_spec(D, 40),
              _full_spec(1, 40)],
    out_specs=_row_spec(40),
    out_shape=jax.ShapeDtypeStruct((N, 40), jnp.float32),
)


def kernel(x, edge_index, W1, b1, W2, b2, Wlin, blin):
    ei = edge_index.astype(jnp.int32)
    src2 = ei[0].reshape(NCH, CHUNK)
    dst2 = ei[1].reshape(NCH, CHUNK)

    degp = _deg_kernel(dst2)
    d0, d1 = degp[:N], degp[N:]

    xs1 = _mm_scale(x, W1, d0, d1)
    a1 = _scatter_kernel(xs1, src2, dst2)
    xs2 = _combine_mm(a1[:N], a1[N:], xs1, d0, d1, b1.reshape(1, D), W2)
    a2 = _scatter_kernel(xs2, src2, dst2)
    out = _final_mm(a2[:N], a2[N:], xs2, d0, d1, b2.reshape(1, D),
                    Wlin.T, blin.reshape(1, 40))
    return out


# speedup vs baseline: 8.0781x; 8.0781x over previous
import functools

import jax
import jax.numpy as jnp
from jax import lax
from jax.experimental import pallas as pl
from jax.experimental.pallas import tpu as pltpu
from jax.experimental.pallas import tpu_sc as plsc

N = 10000
D = 128
E = 320000
CHUNK = 128
NC = 2
NS = 16
CH_PER_TILE = 80
NCH = NC * NS * CH_PER_TILE
E_PAD = NCH * CHUNK
CH_PER_SC = NCH // NC
N_PAD = 10240
ROWS_PER_TILE = N_PAD // NS
DUMMY_ROW = N

_MESH = plsc.VectorSubcoreMesh(core_axis_name="c", subcore_axis_name="s")


def _fill_const(ref, nrows, ncols, val):
    v = jnp.full((16,), val, jnp.float32)

    def body(i, _):
        for k in range(ncols // 16):
            ref[i, pl.ds(k * 16, 16)] = v
        return 0

    lax.fori_loop(0, nrows, body, 0)


@functools.partial(
    pl.kernel,
    out_type=jax.ShapeDtypeStruct((2 * N_PAD, 16), jnp.float32),
    mesh=_MESH,
    compiler_params=pltpu.CompilerParams(use_tc_tiling_on_sc=False),
    scratch_types=[
        pltpu.VMEM((CHUNK, 16), jnp.float32),
        pltpu.VMEM((8, CHUNK), jnp.int32),
        pltpu.VMEM((ROWS_PER_TILE, 16), jnp.float32),
        pltpu.VMEM_SHARED((N_PAD, 16), jnp.float32),
    ],
)
def _deg_kernel(dst_hbm, out_hbm, ones_v, dst_v, zero_v, acc_sh):
    c = lax.axis_index("c")
    s = lax.axis_index("s")

    _fill_const(ones_v, CHUNK, 16, 1.0)
    _fill_const(zero_v, ROWS_PER_TILE, 16, 0.0)

    pltpu.sync_copy(zero_v, acc_sh.at[pl.ds(s * ROWS_PER_TILE, ROWS_PER_TILE)])
    plsc.subcore_barrier()

    base = c * CH_PER_SC + s * CH_PER_TILE

    def body(j, _):
        pltpu.sync_copy(dst_hbm.at[pl.ds(base + j * 8, 8)], dst_v)
        for k in range(8):
            pltpu.sync_copy(ones_v, acc_sh.at[dst_v.at[k]], add=True)
        return 0

    lax.fori_loop(0, CH_PER_TILE // 8, body, 0)

    plsc.subcore_barrier()
    r0 = s * ROWS_PER_TILE
    pltpu.sync_copy(
        acc_sh.at[pl.ds(r0, ROWS_PER_TILE)],
        out_hbm.at[pl.ds(c * N_PAD + r0, ROWS_PER_TILE)],
    )


ZROWS = 128


@functools.partial(
    pl.kernel,
    out_type=jax.ShapeDtypeStruct((2 * N_PAD, D), jnp.float32),
    mesh=_MESH,
    scratch_types=[
        pltpu.VMEM((CHUNK, D), jnp.float32),
        pltpu.VMEM((8, CHUNK), jnp.int32),
        pltpu.VMEM((8, CHUNK), jnp.int32),
        pltpu.VMEM((ZROWS, D), jnp.float32),
        pltpu.VMEM_SHARED((N_PAD, D), jnp.float32),
        pltpu.SemaphoreType.DMA,
    ],
)
def _scatter_kernel(xs_hbm, src_hbm, dst_hbm, out_hbm,
                    rows_v, src_v, dst_v, zero_v, acc_sh, sem):
    c = lax.axis_index("c")
    s = lax.axis_index("s")

    _fill_const(zero_v, ZROWS, D, 0.0)
    for k in range(ROWS_PER_TILE // ZROWS):
        pltpu.sync_copy(
            zero_v, acc_sh.at[pl.ds(s * ROWS_PER_TILE + k * ZROWS, ZROWS)])
    plsc.subcore_barrier()

    base = c * CH_PER_SC + s * CH_PER_TILE

    def body(j, _):
        pltpu.sync_copy(src_hbm.at[pl.ds(base + j * 8, 8)], src_v)
        pltpu.sync_copy(dst_hbm.at[pl.ds(base + j * 8, 8)], dst_v)
        for k in range(8):
            pltpu.async_copy(xs_hbm.at[src_v.at[k]], rows_v, sem).wait()
            pltpu.sync_copy(rows_v, acc_sh.at[dst_v.at[k]], add=True)
        return 0

    lax.fori_loop(0, CH_PER_TILE // 8, body, 0)

    plsc.subcore_barrier()
    r0 = s * ROWS_PER_TILE
    pltpu.sync_copy(
        acc_sh.at[pl.ds(r0, ROWS_PER_TILE)],
        out_hbm.at[pl.ds(c * N_PAD + r0, ROWS_PER_TILE)],
    )


MBLK = 1000
GRID = N // MBLK


def _dis(d0_ref, d1_ref):
    deg = d0_ref[:, 0:1] + d1_ref[:, 0:1] + 1.0
    return lax.rsqrt(deg)


def _mm_scale_body(x_ref, w_ref, d0_ref, d1_ref, o_ref):
    dis = _dis(d0_ref, d1_ref)
    xw = jnp.dot(x_ref[...], w_ref[...], preferred_element_type=jnp.float32)
    o_ref[...] = xw * dis


def _combine_mm_body(a0_ref, a1_ref, xs_ref, d0_ref, d1_ref, b_ref, w_ref,
                     o_ref):
    dis = _dis(d0_ref, d1_ref)
    tot = (a0_ref[...] + a1_ref[...] + xs_ref[...]) * dis + b_ref[...]
    h = jnp.maximum(tot, 0.0)
    o_ref[...] = jnp.dot(h, w_ref[...], preferred_element_type=jnp.float32) * dis


def _final_body(a0_ref, a1_ref, xs_ref, d0_ref, d1_ref, b_ref, wl_ref, bl_ref,
                o_ref):
    dis = _dis(d0_ref, d1_ref)
    tot = (a0_ref[...] + a1_ref[...] + xs_ref[...]) * dis + b_ref[...]
    h = jnp.maximum(tot, 0.0)
    o_ref[...] = (
        jnp.dot(h, wl_ref[...], preferred_element_type=jnp.float32) + bl_ref[...]
    )


def _row_spec(cols):
    return pl.BlockSpec((MBLK, cols), lambda i: (i, 0))


def _full_spec(r, cols):
    return pl.BlockSpec((r, cols), lambda i: (0, 0))


_mm_scale = pl.pallas_call(
    _mm_scale_body,
    grid=(GRID,),
    in_specs=[_row_spec(D), _full_spec(D, D), _row_spec(16), _row_spec(16)],
    out_specs=_row_spec(D),
    out_shape=jax.ShapeDtypeStruct((N, D), jnp.float32),
)

_combine_mm = pl.pallas_call(
    _combine_mm_body,
    grid=(GRID,),
    in_specs=[_row_spec(D), _row_spec(D), _row_spec(D), _row_spec(16),
              _row_spec(16), _full_spec(1, D), _full_spec(D, D)],
    out_specs=_row_spec(D),
    out_shape=jax.ShapeDtypeStruct((N, D), jnp.float32),
)

_final_mm = pl.pallas_call(
    _final_body,
    grid=(GRID,),
    in_specs=[_row_spec(D), _row_spec(D), _row_spec(D), _row_spec(16),
              _row_spec(16), _full_spec(1, D), _full_spec(D, 40),
              _full_spec(1, 40)],
    out_specs=_row_spec(40),
    out_shape=jax.ShapeDtypeStruct((N, 40), jnp.float32),
)


def kernel(x, edge_index, W1, b1, W2, b2, Wlin, blin):
    ei = edge_index.astype(jnp.int32)
    pad = E_PAD - E
    src2 = jnp.concatenate(
        [ei[0], jnp.zeros((pad,), jnp.int32)]).reshape(NCH, CHUNK)
    dst2 = jnp.concatenate(
        [ei[1], jnp.full((pad,), DUMMY_ROW, jnp.int32)]).reshape(NCH, CHUNK)

    degp = _deg_kernel(dst2)
    d0, d1 = degp[:N], degp[N_PAD:N_PAD + N]

    xs1 = _mm_scale(x, W1, d0, d1)
    a1 = _scatter_kernel(xs1, src2, dst2)
    xs2 = _combine_mm(a1[:N], a1[N_PAD:N_PAD + N], xs1, d0, d1,
                      b1.reshape(1, D), W2)
    a2 = _scatter_kernel(xs2, src2, dst2)
    out = _final_mm(a2[:N], a2[N_PAD:N_PAD + N], xs2, d0, d1,
                    b2.reshape(1, D), Wlin.T, blin.reshape(1, 40))
    return out


# speedup vs baseline: 8.8488x; 1.0954x over previous
import functools

import jax
import jax.numpy as jnp
from jax import lax
from jax.experimental import pallas as pl
from jax.experimental.pallas import tpu as pltpu
from jax.experimental.pallas import tpu_sc as plsc

N = 10000
D = 128
E = 320000
CHUNK = 128
NC = 2
NS = 16
CH_PER_TILE = 80
NCH = NC * NS * CH_PER_TILE
E_PAD = NCH * CHUNK
CH_PER_SC = NCH // NC
N_PAD = 10240
ROWS_PER_TILE = N_PAD // NS
DUMMY_ROW = N

_MESH = plsc.VectorSubcoreMesh(core_axis_name="c", subcore_axis_name="s")


def _fill_const(ref, nrows, ncols, val):
    v = jnp.full((16,), val, jnp.float32)

    def body(i, _):
        for k in range(ncols // 16):
            ref[i, pl.ds(k * 16, 16)] = v
        return 0

    lax.fori_loop(0, nrows, body, 0)


@functools.partial(
    pl.kernel,
    out_type=jax.ShapeDtypeStruct((2 * N_PAD, 16), jnp.float32),
    mesh=_MESH,
    compiler_params=pltpu.CompilerParams(use_tc_tiling_on_sc=False),
    scratch_types=[
        pltpu.VMEM((CHUNK, 16), jnp.float32),
        pltpu.VMEM((8, CHUNK), jnp.int32),
        pltpu.VMEM((ROWS_PER_TILE, 16), jnp.float32),
        pltpu.VMEM_SHARED((N_PAD, 16), jnp.float32),
    ],
)
def _deg_kernel(dst_hbm, out_hbm, ones_v, dst_v, zero_v, acc_sh):
    c = lax.axis_index("c")
    s = lax.axis_index("s")

    _fill_const(ones_v, CHUNK, 16, 1.0)
    _fill_const(zero_v, ROWS_PER_TILE, 16, 0.0)

    pltpu.sync_copy(zero_v, acc_sh.at[pl.ds(s * ROWS_PER_TILE, ROWS_PER_TILE)])
    plsc.subcore_barrier()

    base = c * CH_PER_SC + s * CH_PER_TILE

    def body(j, _):
        pltpu.sync_copy(dst_hbm.at[pl.ds(base + j * 8, 8)], dst_v)
        for k in range(8):
            pltpu.sync_copy(ones_v, acc_sh.at[dst_v.at[k]], add=True)
        return 0

    lax.fori_loop(0, CH_PER_TILE // 8, body, 0)

    plsc.subcore_barrier()
    r0 = s * ROWS_PER_TILE
    pltpu.sync_copy(
        acc_sh.at[pl.ds(r0, ROWS_PER_TILE)],
        out_hbm.at[pl.ds(c * N_PAD + r0, ROWS_PER_TILE)],
    )


IBLK = 16
NIB = CH_PER_TILE // IBLK


@functools.partial(
    pl.kernel,
    out_type=jax.ShapeDtypeStruct((2 * N_PAD, D), jnp.float32),
    mesh=_MESH,
    scratch_types=(
        [pltpu.VMEM((CHUNK, D), jnp.float32)] * 2 +
        [pltpu.VMEM((IBLK, CHUNK), jnp.int32)] * 4 +
        [pltpu.VMEM_SHARED((N_PAD, D), jnp.float32)] +
        [pltpu.SemaphoreType.DMA] * 4
    ),
)
def _scatter_kernel(xs_hbm, src_hbm, dst_hbm, out_hbm,
                    r0, r1, sa, sb, da, db, acc_sh, g0, g1, s0, s1):
    c = lax.axis_index("c")
    s = lax.axis_index("s")
    rows = [r0, r1]
    srcb = [sa, sb]
    dstb = [da, db]
    gsem = [g0, g1]
    ssem = [s0, s1]

    _fill_const(r0, CHUNK, D, 0.0)
    for k in range(ROWS_PER_TILE // CHUNK):
        pltpu.sync_copy(
            r0, acc_sh.at[pl.ds(s * ROWS_PER_TILE + k * CHUNK, CHUNK)])
    plsc.subcore_barrier()

    base = c * CH_PER_SC + s * CH_PER_TILE

    def load_iblk(m):
        pltpu.sync_copy(src_hbm.at[pl.ds(base + m * IBLK, IBLK)], srcb[m % 2])
        pltpu.sync_copy(dst_hbm.at[pl.ds(base + m * IBLK, IBLK)], dstb[m % 2])

    def gath(ci, k):
        pltpu.async_copy(
            xs_hbm.at[srcb[(ci // IBLK) % 2].at[ci % IBLK]], rows[k], gsem[k])

    def wait_g(k):
        pltpu.make_async_copy(
            xs_hbm.at[pl.ds(0, CHUNK)], rows[k], gsem[k]).wait()

    def scat(ci, k):
        pltpu.async_copy(
            rows[k], acc_sh.at[dstb[(ci // IBLK) % 2].at[ci % IBLK]],
            ssem[k], add=True)

    def wait_s(k):
        pltpu.make_async_copy(
            rows[k], acc_sh.at[pl.ds(0, CHUNK)], ssem[k]).wait()

    load_iblk(0)
    gath(0, 0)
    for m in range(NIB):
        if m > 0:
            wait_s(1)
        if m + 1 < NIB:
            load_iblk(m + 1)
        for k in range(IBLK):
            ci = m * IBLK + k
            sl = ci % 2
            wait_g(sl)
            scat(ci, sl)
            if ci + 1 < CH_PER_TILE:
                if k > 0:
                    wait_s(1 - sl)
                gath(ci + 1, 1 - sl)
    wait_s(0)
    wait_s(1)

    plsc.subcore_barrier()
    r0 = s * ROWS_PER_TILE
    pltpu.sync_copy(
        acc_sh.at[pl.ds(r0, ROWS_PER_TILE)],
        out_hbm.at[pl.ds(c * N_PAD + r0, ROWS_PER_TILE)],
    )


MBLK = 1000
GRID = N // MBLK


def _dis(d0_ref, d1_ref):
    deg = d0_ref[:, 0:1] + d1_ref[:, 0:1] + 1.0
    return lax.rsqrt(deg)


def _mm_scale_body(x_ref, w_ref, d0_ref, d1_ref, o_ref):
    dis = _dis(d0_ref, d1_ref)
    xw = jnp.dot(x_ref[...], w_ref[...], preferred_element_type=jnp.float32)
    o_ref[...] = xw * dis


def _combine_mm_body(a0_ref, a1_ref, xs_ref, d0_ref, d1_ref, b_ref, w_ref,
                     o_ref):
    dis = _dis(d0_ref, d1_ref)
    tot = (a0_ref[...] + a1_ref[...] + xs_ref[...]) * dis + b_ref[...]
    h = jnp.maximum(tot, 0.0)
    o_ref[...] = jnp.dot(h, w_ref[...], preferred_element_type=jnp.float32) * dis


def _final_body(a0_ref, a1_ref, xs_ref, d0_ref, d1_ref, b_ref, wl_ref, bl_ref,
                o_ref):
    dis = _dis(d0_ref, d1_ref)
    tot = (a0_ref[...] + a1_ref[...] + xs_ref[...]) * dis + b_ref[...]
    h = jnp.maximum(tot, 0.0)
    o_ref[...] = (
        jnp.dot(h, wl_ref[...], preferred_element_type=jnp.float32) + bl_ref[...]
    )


def _row_spec(cols):
    return pl.BlockSpec((MBLK, cols), lambda i: (i, 0))


def _full_spec(r, cols):
    return pl.BlockSpec((r, cols), lambda i: (0, 0))


_mm_scale = pl.pallas_call(
    _mm_scale_body,
    grid=(GRID,),
    in_specs=[_row_spec(D), _full_spec(D, D), _row_spec(16), _row_spec(16)],
    out_specs=_row_spec(D),
    out_shape=jax.ShapeDtypeStruct((N, D), jnp.float32),
)

_combine_mm = pl.pallas_call(
    _combine_mm_body,
    grid=(GRID,),
    in_specs=[_row_spec(D), _row_spec(D), _row_spec(D), _row_spec(16),
              _row_spec(16), _full_spec(1, D), _full_spec(D, D)],
    out_specs=_row_spec(D),
    out_shape=jax.ShapeDtypeStruct((N, D), jnp.float32),
)

_final_mm = pl.pallas_call(
    _final_body,
    grid=(GRID,),
    in_specs=[_row_spec(D), _row_spec(D), _row_spec(D), _row_spec(16),
              _row_spec(16), _full_spec(1, D), _full_spec(D, 40),
              _full_spec(1, 40)],
    out_specs=_row_spec(40),
    out_shape=jax.ShapeDtypeStruct((N, 40), jnp.float32),
)


def kernel(x, edge_index, W1, b1, W2, b2, Wlin, blin):
    ei = edge_index.astype(jnp.int32)
    pad = E_PAD - E
    src2 = jnp.concatenate(
        [ei[0], jnp.zeros((pad,), jnp.int32)]).reshape(NCH, CHUNK)
    dst2 = jnp.concatenate(
        [ei[1], jnp.full((pad,), DUMMY_ROW, jnp.int32)]).reshape(NCH, CHUNK)

    degp = _deg_kernel(dst2)
    d0, d1 = degp[:N], degp[N_PAD:N_PAD + N]

    xs1 = _mm_scale(x, W1, d0, d1)
    a1 = _scatter_kernel(xs1, src2, dst2)
    xs2 = _combine_mm(a1[:N], a1[N_PAD:N_PAD + N], xs1, d0, d1,
                      b1.reshape(1, D), W2)
    a2 = _scatter_kernel(xs2, src2, dst2)
    out = _final_mm(a2[:N], a2[N_PAD:N_PAD + N], xs2, d0, d1,
                    b2.reshape(1, D), Wlin.T, blin.reshape(1, 40))
    return out


# speedup vs baseline: 20.9644x; 2.3692x over previous
import functools

import jax
import jax.numpy as jnp
from jax import lax
from jax.experimental import pallas as pl
from jax.experimental.pallas import tpu as pltpu
from jax.experimental.pallas import tpu_sc as plsc

N = 10000
D = 128
F = D // 2
E = 320000
CHUNK = 128
NC = 2
NS = 16
CH_PER_TILE = 160
NCH = NS * CH_PER_TILE
E_PAD = NCH * CHUNK
N_PAD = 10240
ROWS_PER_TILE = N_PAD // NS
DUMMY_ROW = N

_MESH = plsc.VectorSubcoreMesh(core_axis_name="c", subcore_axis_name="s")


def _fill_const(ref, nrows, ncols, val):
    v = jnp.full((16,), val, jnp.float32)

    def body(i, _):
        for k in range(ncols // 16):
            ref[i, pl.ds(k * 16, 16)] = v
        return 0

    lax.fori_loop(0, nrows, body, 0)


DEG_CH_PER_TILE = CH_PER_TILE // 2


@functools.partial(
    pl.kernel,
    out_type=jax.ShapeDtypeStruct((2 * N_PAD, 16), jnp.float32),
    mesh=_MESH,
    compiler_params=pltpu.CompilerParams(use_tc_tiling_on_sc=False),
    scratch_types=[
        pltpu.VMEM((CHUNK, 16), jnp.float32),
        pltpu.VMEM((8, CHUNK), jnp.int32),
        pltpu.VMEM((ROWS_PER_TILE, 16), jnp.float32),
        pltpu.VMEM_SHARED((N_PAD, 16), jnp.float32),
    ],
)
def _deg_kernel(dst_hbm, out_hbm, ones_v, dst_v, zero_v, acc_sh):
    c = lax.axis_index("c")
    s = lax.axis_index("s")

    _fill_const(ones_v, CHUNK, 16, 1.0)
    _fill_const(zero_v, ROWS_PER_TILE, 16, 0.0)

    pltpu.sync_copy(zero_v, acc_sh.at[pl.ds(s * ROWS_PER_TILE, ROWS_PER_TILE)])
    plsc.subcore_barrier()

    base = (c * NS + s) * DEG_CH_PER_TILE

    def body(j, _):
        pltpu.sync_copy(dst_hbm.at[pl.ds(base + j * 8, 8)], dst_v)
        for k in range(8):
            pltpu.sync_copy(ones_v, acc_sh.at[dst_v.at[k]], add=True)
        return 0

    lax.fori_loop(0, DEG_CH_PER_TILE // 8, body, 0)

    plsc.subcore_barrier()
    r0 = s * ROWS_PER_TILE
    pltpu.sync_copy(
        acc_sh.at[pl.ds(r0, ROWS_PER_TILE)],
        out_hbm.at[pl.ds(c * N_PAD + r0, ROWS_PER_TILE)],
    )


IBLK = 16
NIB = CH_PER_TILE // IBLK
STAGE = 640


@functools.partial(
    pl.kernel,
    out_type=jax.ShapeDtypeStruct((2 * N_PAD, F), jnp.float32),
    mesh=_MESH,
    compiler_params=pltpu.CompilerParams(use_tc_tiling_on_sc=False),
    scratch_types=(
        [pltpu.VMEM((CHUNK, F), jnp.float32)] * 2 +
        [pltpu.VMEM((IBLK, CHUNK), jnp.int32)] * 4 +
        [
            pltpu.VMEM_SHARED((N, F), jnp.float32),
            pltpu.VMEM_SHARED((N_PAD, F), jnp.float32),
        ] +
        [pltpu.SemaphoreType.DMA] * 4
    ),
)
def _scatter_kernel(xs_lo_hbm, xs_hi_hbm, src_hbm, dst_hbm, out_hbm,
                    r0, r1, sa, sb, da, db, xs_sp, acc_sh, g0, g1, s0, s1):
    c = lax.axis_index("c")
    s = lax.axis_index("s")
    rows = [r0, r1]
    srcb = [sa, sb]
    dstb = [da, db]
    gsem = [g0, g1]
    ssem = [s0, s1]

    _fill_const(r0, CHUNK, F, 0.0)
    for k in range(ROWS_PER_TILE // CHUNK):
        pltpu.sync_copy(
            r0, acc_sh.at[pl.ds(s * ROWS_PER_TILE + k * CHUNK, CHUNK)])

    def stage(src_half):
        @pl.when(s < NS - 1)
        def _():
            pltpu.sync_copy(src_half.at[pl.ds(s * STAGE, STAGE)],
                            xs_sp.at[pl.ds(s * STAGE, STAGE)])

        @pl.when(s == NS - 1)
        def _():
            pltpu.sync_copy(src_half.at[pl.ds((NS - 1) * STAGE, N - (NS - 1) * STAGE)],
                            xs_sp.at[pl.ds((NS - 1) * STAGE, N - (NS - 1) * STAGE)])

    @pl.when(c == 0)
    def _():
        stage(xs_lo_hbm)

    @pl.when(c == 1)
    def _():
        stage(xs_hi_hbm)

    plsc.subcore_barrier()

    base = s * CH_PER_TILE

    def load_iblk(m):
        pltpu.sync_copy(src_hbm.at[pl.ds(base + m * IBLK, IBLK)], srcb[m % 2])
        pltpu.sync_copy(dst_hbm.at[pl.ds(base + m * IBLK, IBLK)], dstb[m % 2])

    def gath(ci, k):
        pltpu.async_copy(
            xs_sp.at[srcb[(ci // IBLK) % 2].at[ci % IBLK]], rows[k], gsem[k])

    def wait_g(k):
        pltpu.make_async_copy(
            xs_sp.at[pl.ds(0, CHUNK)], rows[k], gsem[k]).wait()

    def scat(ci, k):
        pltpu.async_copy(
            rows[k], acc_sh.at[dstb[(ci // IBLK) % 2].at[ci % IBLK]],
            ssem[k], add=True)

    def wait_s(k):
        pltpu.make_async_copy(
            rows[k], acc_sh.at[pl.ds(0, CHUNK)], ssem[k]).wait()

    load_iblk(0)
    gath(0, 0)
    for m in range(NIB):
        if m > 0:
            wait_s(1)
        if m + 1 < NIB:
            load_iblk(m + 1)
        for k in range(IBLK):
            ci = m * IBLK + k
            sl = ci % 2
            wait_g(sl)
            scat(ci, sl)
            if ci + 1 < CH_PER_TILE:
                if k > 0:
                    wait_s(1 - sl)
                gath(ci + 1, 1 - sl)
    wait_s(0)
    wait_s(1)

    plsc.subcore_barrier()
    r0o = s * ROWS_PER_TILE
    pltpu.sync_copy(
        acc_sh.at[pl.ds(r0o, ROWS_PER_TILE)],
        out_hbm.at[pl.ds(c * N_PAD + r0o, ROWS_PER_TILE)],
    )


MBLK = 1000
GRID = N // MBLK


def _dis(d0_ref, d1_ref):
    deg = d0_ref[:, 0:1] + d1_ref[:, 0:1] + 1.0
    return lax.rsqrt(deg)


def _mm_scale_body(x_ref, w_ref, d0_ref, d1_ref, lo_ref, hi_ref):
    dis = _dis(d0_ref, d1_ref)
    xw = jnp.dot(x_ref[...], w_ref[...], preferred_element_type=jnp.float32)
    xs = xw * dis
    lo_ref[...] = xs[:, :F]
    hi_ref[...] = xs[:, F:]


def _combine_mm_body(alo_ref, ahi_ref, xlo_ref, xhi_ref, d0_ref, d1_ref,
                     b_ref, w_ref, lo_ref, hi_ref):
    dis = _dis(d0_ref, d1_ref)
    tot = jnp.concatenate(
        [alo_ref[...] + xlo_ref[...], ahi_ref[...] + xhi_ref[...]], axis=1)
    h = jnp.maximum(tot * dis + b_ref[...], 0.0)
    xs = jnp.dot(h, w_ref[...], preferred_element_type=jnp.float32) * dis
    lo_ref[...] = xs[:, :F]
    hi_ref[...] = xs[:, F:]


def _final_body(alo_ref, ahi_ref, xlo_ref, xhi_ref, d0_ref, d1_ref, b_ref,
                wl_ref, bl_ref, o_ref):
    dis = _dis(d0_ref, d1_ref)
    tot = jnp.concatenate(
        [alo_ref[...] + xlo_ref[...], ahi_ref[...] + xhi_ref[...]], axis=1)
    h = jnp.maximum(tot * dis + b_ref[...], 0.0)
    o_ref[...] = (
        jnp.dot(h, wl_ref[...], preferred_element_type=jnp.float32) + bl_ref[...]
    )


def _row_spec(cols):
    return pl.BlockSpec((MBLK, cols), lambda i: (i, 0))


def _full_spec(r, cols):
    return pl.BlockSpec((r, cols), lambda i: (0, 0))


_mm_scale = pl.pallas_call(
    _mm_scale_body,
    grid=(GRID,),
    in_specs=[_row_spec(D), _full_spec(D, D), _row_spec(16), _row_spec(16)],
    out_specs=[_row_spec(F), _row_spec(F)],
    out_shape=[jax.ShapeDtypeStruct((N, F), jnp.float32)] * 2,
)

_combine_mm = pl.pallas_call(
    _combine_mm_body,
    grid=(GRID,),
    in_specs=[_row_spec(F), _row_spec(F), _row_spec(F), _row_spec(F),
              _row_spec(16), _row_spec(16), _full_spec(1, D),
              _full_spec(D, D)],
    out_specs=[_row_spec(F), _row_spec(F)],
    out_shape=[jax.ShapeDtypeStruct((N, F), jnp.float32)] * 2,
)

_final_mm = pl.pallas_call(
    _final_body,
    grid=(GRID,),
    in_specs=[_row_spec(F), _row_spec(F), _row_spec(F), _row_spec(F),
              _row_spec(16), _row_spec(16), _full_spec(1, D),
              _full_spec(D, 40), _full_spec(1, 40)],
    out_specs=_row_spec(40),
    out_shape=jax.ShapeDtypeStruct((N, 40), jnp.float32),
)


def kernel(x, edge_index, W1, b1, W2, b2, Wlin, blin):
    ei = edge_index.astype(jnp.int32)
    pad = E_PAD - E
    src2 = jnp.concatenate(
        [ei[0], jnp.zeros((pad,), jnp.int32)]).reshape(NCH, CHUNK)
    dst2 = jnp.concatenate(
        [ei[1], jnp.full((pad,), DUMMY_ROW, jnp.int32)]).reshape(NCH, CHUNK)

    degp = _deg_kernel(dst2)
    d0, d1 = degp[:N], degp[N_PAD:N_PAD + N]

    x1lo, x1hi = _mm_scale(x, W1, d0, d1)
    a1 = _scatter_kernel(x1lo, x1hi, src2, dst2)
    x2lo, x2hi = _combine_mm(a1[:N], a1[N_PAD:N_PAD + N], x1lo, x1hi, d0, d1,
                             b1.reshape(1, D), W2)
    a2 = _scatter_kernel(x2lo, x2hi, src2, dst2)
    out = _final_mm(a2[:N], a2[N_PAD:N_PAD + N], x2lo, x2hi, d0, d1,
                    b2.reshape(1, D), Wlin.T, blin.reshape(1, 40))
    return out


# speedup vs baseline: 22.5297x; 1.0747x over previous
import functools

import jax
import jax.numpy as jnp
from jax import lax
from jax.experimental import pallas as pl
from jax.experimental.pallas import tpu as pltpu
from jax.experimental.pallas import tpu_sc as plsc

N = 10000
D = 128
F = D // 2
E = 320000
CHUNK = 128
NC = 2
NS = 16
CH_PER_TILE = 160
NCH = NS * CH_PER_TILE
E_PAD = NCH * CHUNK
N_PAD = 10240
ROWS_PER_TILE = N_PAD // NS
TAIL = N - (NS - 1) * ROWS_PER_TILE
DUMMY_ROW = N

_MESH = plsc.VectorSubcoreMesh(core_axis_name="c", subcore_axis_name="s")


def _fill_const(ref, nrows, ncols, val):
    v = jnp.full((16,), val, jnp.float32)

    def body(i, _):
        for k in range(ncols // 16):
            ref[i, pl.ds(k * 16, 16)] = v
        return 0

    lax.fori_loop(0, nrows, body, 0)


def _copy_out(acc_sh, dst_ref, s):
    @pl.when(s < NS - 1)
    def _():
        pltpu.sync_copy(acc_sh.at[pl.ds(s * ROWS_PER_TILE, ROWS_PER_TILE)],
                        dst_ref.at[pl.ds(s * ROWS_PER_TILE, ROWS_PER_TILE)])

    @pl.when(s == NS - 1)
    def _():
        r0 = (NS - 1) * ROWS_PER_TILE
        pltpu.sync_copy(acc_sh.at[pl.ds(r0, TAIL)],
                        dst_ref.at[pl.ds(r0, TAIL)])


DEG_CH_PER_TILE = CH_PER_TILE // 2


@functools.partial(
    pl.kernel,
    out_type=[jax.ShapeDtypeStruct((N, 16), jnp.float32)] * 2,
    mesh=_MESH,
    compiler_params=pltpu.CompilerParams(use_tc_tiling_on_sc=False),
    scratch_types=[
        pltpu.VMEM((CHUNK, 16), jnp.float32),
        pltpu.VMEM((8, CHUNK), jnp.int32),
        pltpu.VMEM((ROWS_PER_TILE, 16), jnp.float32),
        pltpu.VMEM_SHARED((N_PAD, 16), jnp.float32),
    ],
)
def _deg_kernel(ei_hbm, out0_hbm, out1_hbm, ones_v, dst_v, zero_v, acc_sh):
    c = lax.axis_index("c")
    s = lax.axis_index("s")

    _fill_const(ones_v, CHUNK, 16, 1.0)
    _fill_const(zero_v, ROWS_PER_TILE, 16, 0.0)

    pltpu.sync_copy(zero_v, acc_sh.at[pl.ds(s * ROWS_PER_TILE, ROWS_PER_TILE)])
    plsc.subcore_barrier()

    base = (c * NS + s) * DEG_CH_PER_TILE

    def body(j, _):
        pltpu.sync_copy(ei_hbm.at[1].at[pl.ds(base + j * 8, 8)], dst_v)
        for k in range(8):
            pltpu.sync_copy(ones_v, acc_sh.at[dst_v.at[k]], add=True)
        return 0

    lax.fori_loop(0, DEG_CH_PER_TILE // 8, body, 0)

    plsc.subcore_barrier()

    @pl.when(c == 0)
    def _():
        _copy_out(acc_sh, out0_hbm, s)

    @pl.when(c == 1)
    def _():
        _copy_out(acc_sh, out1_hbm, s)


IBLK = 16
NIB = CH_PER_TILE // IBLK
STAGE = 640


@functools.partial(
    pl.kernel,
    out_type=[jax.ShapeDtypeStruct((N, F), jnp.float32)] * 2,
    mesh=_MESH,
    compiler_params=pltpu.CompilerParams(use_tc_tiling_on_sc=False),
    scratch_types=(
        [pltpu.VMEM((CHUNK, F), jnp.float32)] * 2 +
        [pltpu.VMEM((IBLK, CHUNK), jnp.int32)] * 4 +
        [
            pltpu.VMEM_SHARED((N, F), jnp.float32),
            pltpu.VMEM_SHARED((N_PAD, F), jnp.float32),
        ] +
        [pltpu.SemaphoreType.DMA] * 4
    ),
)
def _scatter_kernel(xs_lo_hbm, xs_hi_hbm, ei_hbm, out0_hbm, out1_hbm,
                    r0, r1, sa, sb, da, db, xs_sp, acc_sh, g0, g1, s0, s1):
    c = lax.axis_index("c")
    s = lax.axis_index("s")
    rows = [r0, r1]
    srcb = [sa, sb]
    dstb = [da, db]
    gsem = [g0, g1]
    ssem = [s0, s1]

    _fill_const(r0, CHUNK, F, 0.0)
    for k in range(ROWS_PER_TILE // CHUNK):
        pltpu.sync_copy(
            r0, acc_sh.at[pl.ds(s * ROWS_PER_TILE + k * CHUNK, CHUNK)])

    def stage(src_half):
        @pl.when(s < NS - 1)
        def _():
            pltpu.sync_copy(src_half.at[pl.ds(s * STAGE, STAGE)],
                            xs_sp.at[pl.ds(s * STAGE, STAGE)])

        @pl.when(s == NS - 1)
        def _():
            pltpu.sync_copy(src_half.at[pl.ds((NS - 1) * STAGE, TAIL)],
                            xs_sp.at[pl.ds((NS - 1) * STAGE, TAIL)])

    @pl.when(c == 0)
    def _():
        stage(xs_lo_hbm)

    @pl.when(c == 1)
    def _():
        stage(xs_hi_hbm)

    plsc.subcore_barrier()

    base = s * CH_PER_TILE

    def load_iblk(m):
        pltpu.sync_copy(
            ei_hbm.at[0].at[pl.ds(base + m * IBLK, IBLK)], srcb[m % 2])
        pltpu.sync_copy(
            ei_hbm.at[1].at[pl.ds(base + m * IBLK, IBLK)], dstb[m % 2])

    def gath(ci, k):
        pltpu.async_copy(
            xs_sp.at[srcb[(ci // IBLK) % 2].at[ci % IBLK]], rows[k], gsem[k])

    def wait_g(k):
        pltpu.make_async_copy(
            xs_sp.at[pl.ds(0, CHUNK)], rows[k], gsem[k]).wait()

    def scat(ci, k):
        pltpu.async_copy(
            rows[k], acc_sh.at[dstb[(ci // IBLK) % 2].at[ci % IBLK]],
            ssem[k], add=True)

    def wait_s(k):
        pltpu.make_async_copy(
            rows[k], acc_sh.at[pl.ds(0, CHUNK)], ssem[k]).wait()

    load_iblk(0)
    gath(0, 0)
    for m in range(NIB):
        if m > 0:
            wait_s(1)
        if m + 1 < NIB:
            load_iblk(m + 1)
        for k in range(IBLK):
            ci = m * IBLK + k
            sl = ci % 2
            wait_g(sl)
            scat(ci, sl)
            if ci + 1 < CH_PER_TILE:
                if k > 0:
                    wait_s(1 - sl)
                gath(ci + 1, 1 - sl)
    wait_s(0)
    wait_s(1)

    plsc.subcore_barrier()

    @pl.when(c == 0)
    def _():
        _copy_out(acc_sh, out0_hbm, s)

    @pl.when(c == 1)
    def _():
        _copy_out(acc_sh, out1_hbm, s)


MBLK = 1000
GRID = N // MBLK


def _dis(d0_ref, d1_ref):
    deg = d0_ref[:, 0:1] + d1_ref[:, 0:1] + 1.0
    return lax.rsqrt(deg)


def _mm_scale_body(x_ref, w_ref, d0_ref, d1_ref, lo_ref, hi_ref):
    dis = _dis(d0_ref, d1_ref)
    xw = jnp.dot(x_ref[...], w_ref[...], preferred_element_type=jnp.float32)
    xs = xw * dis
    lo_ref[...] = xs[:, :F]
    hi_ref[...] = xs[:, F:]


def _combine_mm_body(alo_ref, ahi_ref, xlo_ref, xhi_ref, d0_ref, d1_ref,
                     b_ref, w_ref, lo_ref, hi_ref):
    dis = _dis(d0_ref, d1_ref)
    tot = jnp.concatenate(
        [alo_ref[...] + xlo_ref[...], ahi_ref[...] + xhi_ref[...]], axis=1)
    h = jnp.maximum(tot * dis + b_ref[...], 0.0)
    xs = jnp.dot(h, w_ref[...], preferred_element_type=jnp.float32) * dis
    lo_ref[...] = xs[:, :F]
    hi_ref[...] = xs[:, F:]


def _final_body(alo_ref, ahi_ref, xlo_ref, xhi_ref, d0_ref, d1_ref, b_ref,
                wl_ref, bl_ref, o_ref):
    dis = _dis(d0_ref, d1_ref)
    tot = jnp.concatenate(
        [alo_ref[...] + xlo_ref[...], ahi_ref[...] + xhi_ref[...]], axis=1)
    h = jnp.maximum(tot * dis + b_ref[...], 0.0)
    o_ref[...] = (
        jnp.dot(h, wl_ref[...], preferred_element_type=jnp.float32) + bl_ref[...]
    )


def _row_spec(cols):
    return pl.BlockSpec((MBLK, cols), lambda i: (i, 0))


def _full_spec(r, cols):
    return pl.BlockSpec((r, cols), lambda i: (0, 0))


_mm_scale = pl.pallas_call(
    _mm_scale_body,
    grid=(GRID,),
    in_specs=[_row_spec(D), _full_spec(D, D), _row_spec(16), _row_spec(16)],
    out_specs=[_row_spec(F), _row_spec(F)],
    out_shape=[jax.ShapeDtypeStruct((N, F), jnp.float32)] * 2,
)

_combine_mm = pl.pallas_call(
    _combine_mm_body,
    grid=(GRID,),
    in_specs=[_row_spec(F), _row_spec(F), _row_spec(F), _row_spec(F),
              _row_spec(16), _row_spec(16), _full_spec(1, D),
              _full_spec(D, D)],
    out_specs=[_row_spec(F), _row_spec(F)],
    out_shape=[jax.ShapeDtypeStruct((N, F), jnp.float32)] * 2,
)

_final_mm = pl.pallas_call(
    _final_body,
    grid=(GRID,),
    in_specs=[_row_spec(F), _row_spec(F), _row_spec(F), _row_spec(F),
              _row_spec(16), _row_spec(16), _full_spec(1, D),
              _full_spec(D, 40), _full_spec(1, 40)],
    out_specs=_row_spec(40),
    out_shape=jax.ShapeDtypeStruct((N, 40), jnp.float32),
)


def kernel(x, edge_index, W1, b1, W2, b2, Wlin, blin):
    ei = edge_index.astype(jnp.int32)
    pad = E_PAD - E
    pad_block = jnp.concatenate(
        [jnp.zeros((1, pad), jnp.int32),
         jnp.full((1, pad), DUMMY_ROW, jnp.int32)], axis=0)
    ei3 = jnp.concatenate([ei, pad_block], axis=1).reshape(2, NCH, CHUNK)

    d0, d1 = _deg_kernel(ei3)

    x1lo, x1hi = _mm_scale(x, W1, d0, d1)
    a1lo, a1hi = _scatter_kernel(x1lo, x1hi, ei3)
    x2lo, x2hi = _combine_mm(a1lo, a1hi, x1lo, x1hi, d0, d1,
                             b1.reshape(1, D), W2)
    a2lo, a2hi = _scatter_kernel(x2lo, x2hi, ei3)
    out = _final_mm(a2lo, a2hi, x2lo, x2hi, d0, d1,
                    b2.reshape(1, D), Wlin.T, blin.reshape(1, 40))
    return out


# speedup vs baseline: 23.3020x; 1.0343x over previous
import functools

import jax
import jax.numpy as jnp
from jax import lax
from jax.experimental import pallas as pl
from jax.experimental.pallas import tpu as pltpu
from jax.experimental.pallas import tpu_sc as plsc

N = 10000
D = 128
F = D // 2
E = 320000
CHUNK = 128
NC = 2
NS = 16
CH_PER_TILE = 160
NCH = NS * CH_PER_TILE
E_PAD = NCH * CHUNK
N_PAD = 10240
ROWS_PER_TILE = N_PAD // NS
TAIL = N - (NS - 1) * ROWS_PER_TILE
DUMMY_ROW = N

_MESH = plsc.VectorSubcoreMesh(core_axis_name="c", subcore_axis_name="s")


def _fill_const(ref, nrows, ncols, val):
    v = jnp.full((16,), val, jnp.float32)

    def body(i, _):
        for k in range(ncols // 16):
            ref[i, pl.ds(k * 16, 16)] = v
        return 0

    lax.fori_loop(0, nrows, body, 0)


def _copy_out(acc_sh, dst_ref, s):
    @pl.when(s < NS - 1)
    def _():
        pltpu.sync_copy(acc_sh.at[pl.ds(s * ROWS_PER_TILE, ROWS_PER_TILE)],
                        dst_ref.at[pl.ds(s * ROWS_PER_TILE, ROWS_PER_TILE)])

    @pl.when(s == NS - 1)
    def _():
        r0 = (NS - 1) * ROWS_PER_TILE
        pltpu.sync_copy(acc_sh.at[pl.ds(r0, TAIL)],
                        dst_ref.at[pl.ds(r0, TAIL)])


DEG_CH_PER_TILE = CH_PER_TILE // 2


@functools.partial(
    pl.kernel,
    out_type=[jax.ShapeDtypeStruct((N, 16), jnp.float32)] * 2,
    mesh=_MESH,
    compiler_params=pltpu.CompilerParams(use_tc_tiling_on_sc=False),
    scratch_types=[
        pltpu.VMEM((CHUNK, 16), jnp.float32),
        pltpu.VMEM((8, CHUNK), jnp.int32),
        pltpu.VMEM((8, CHUNK), jnp.int32),
        pltpu.VMEM((ROWS_PER_TILE, 16), jnp.float32),
        pltpu.VMEM_SHARED((N_PAD, 16), jnp.float32),
        pltpu.SemaphoreType.DMA,
        pltpu.SemaphoreType.DMA,
    ],
)
def _deg_kernel(ei_hbm, out0_hbm, out1_hbm, ones_v, dva, dvb, zero_v, acc_sh,
                sma, smb):
    c = lax.axis_index("c")
    s = lax.axis_index("s")
    dv = [dva, dvb]
    sm = [sma, smb]

    _fill_const(ones_v, CHUNK, 16, 1.0)
    _fill_const(zero_v, ROWS_PER_TILE, 16, 0.0)

    pltpu.sync_copy(zero_v, acc_sh.at[pl.ds(s * ROWS_PER_TILE, ROWS_PER_TILE)])
    plsc.subcore_barrier()

    base = (c * NS + s) * DEG_CH_PER_TILE
    nblk = DEG_CH_PER_TILE // 8

    for m in range(nblk):
        if m >= 2:
            for _ in range(8):
                pltpu.make_async_copy(
                    ones_v, acc_sh.at[pl.ds(0, CHUNK)], sm[m % 2]).wait()
        pltpu.sync_copy(ei_hbm.at[1].at[pl.ds(base + m * 8, 8)], dv[m % 2])
        for k in range(8):
            pltpu.async_copy(
                ones_v, acc_sh.at[dv[m % 2].at[k]], sm[m % 2], add=True)
    for p in range(2):
        for _ in range(8):
            pltpu.make_async_copy(
                ones_v, acc_sh.at[pl.ds(0, CHUNK)], sm[(nblk + p) % 2]).wait()

    plsc.subcore_barrier()

    @pl.when(c == 0)
    def _():
        _copy_out(acc_sh, out0_hbm, s)

    @pl.when(c == 1)
    def _():
        _copy_out(acc_sh, out1_hbm, s)


IBLK = 32
NIB = CH_PER_TILE // IBLK
STAGE = 640


@functools.partial(
    pl.kernel,
    out_type=[jax.ShapeDtypeStruct((N, F), jnp.float32)] * 2,
    mesh=_MESH,
    compiler_params=pltpu.CompilerParams(use_tc_tiling_on_sc=False),
    scratch_types=(
        [pltpu.VMEM((CHUNK, F), jnp.float32)] * 2 +
        [pltpu.VMEM((IBLK, CHUNK), jnp.int32)] * 4 +
        [
            pltpu.VMEM_SHARED((N, F), jnp.float32),
            pltpu.VMEM_SHARED((N_PAD, F), jnp.float32),
        ] +
        [pltpu.SemaphoreType.DMA] * 4
    ),
)
def _scatter_kernel(xs_lo_hbm, xs_hi_hbm, ei_hbm, out0_hbm, out1_hbm,
                    r0, r1, sa, sb, da, db, xs_sp, acc_sh, g0, g1, s0, s1):
    c = lax.axis_index("c")
    s = lax.axis_index("s")
    rows = [r0, r1]
    srcb = [sa, sb]
    dstb = [da, db]
    gsem = [g0, g1]
    ssem = [s0, s1]

    _fill_const(r0, CHUNK, F, 0.0)
    for k in range(ROWS_PER_TILE // CHUNK):
        pltpu.sync_copy(
            r0, acc_sh.at[pl.ds(s * ROWS_PER_TILE + k * CHUNK, CHUNK)])

    def stage(src_half):
        @pl.when(s < NS - 1)
        def _():
            pltpu.sync_copy(src_half.at[pl.ds(s * STAGE, STAGE)],
                            xs_sp.at[pl.ds(s * STAGE, STAGE)])

        @pl.when(s == NS - 1)
        def _():
            pltpu.sync_copy(src_half.at[pl.ds((NS - 1) * STAGE, TAIL)],
                            xs_sp.at[pl.ds((NS - 1) * STAGE, TAIL)])

    @pl.when(c == 0)
    def _():
        stage(xs_lo_hbm)

    @pl.when(c == 1)
    def _():
        stage(xs_hi_hbm)

    plsc.subcore_barrier()

    base = s * CH_PER_TILE

    def load_iblk(m):
        pltpu.sync_copy(
            ei_hbm.at[0].at[pl.ds(base + m * IBLK, IBLK)], srcb[m % 2])
        pltpu.sync_copy(
            ei_hbm.at[1].at[pl.ds(base + m * IBLK, IBLK)], dstb[m % 2])

    def gath(ci, k):
        pltpu.async_copy(
            xs_sp.at[srcb[(ci // IBLK) % 2].at[ci % IBLK]], rows[k], gsem[k])

    def wait_g(k):
        pltpu.make_async_copy(
            xs_sp.at[pl.ds(0, CHUNK)], rows[k], gsem[k]).wait()

    def scat(ci, k):
        pltpu.async_copy(
            rows[k], acc_sh.at[dstb[(ci // IBLK) % 2].at[ci % IBLK]],
            ssem[k], add=True)

    def wait_s(k):
        pltpu.make_async_copy(
            rows[k], acc_sh.at[pl.ds(0, CHUNK)], ssem[k]).wait()

    load_iblk(0)
    gath(0, 0)
    for m in range(NIB):
        if m > 0:
            wait_s(1)
        if m + 1 < NIB:
            load_iblk(m + 1)
        for k in range(IBLK):
            ci = m * IBLK + k
            sl = ci % 2
            wait_g(sl)
            scat(ci, sl)
            if ci + 1 < CH_PER_TILE:
                if k > 0:
                    wait_s(1 - sl)
                gath(ci + 1, 1 - sl)
    wait_s(0)
    wait_s(1)

    plsc.subcore_barrier()

    @pl.when(c == 0)
    def _():
        _copy_out(acc_sh, out0_hbm, s)

    @pl.when(c == 1)
    def _():
        _copy_out(acc_sh, out1_hbm, s)


MBLK = 1000
GRID = N // MBLK


def _dis(d0_ref, d1_ref):
    deg = d0_ref[:, 0:1] + d1_ref[:, 0:1] + 1.0
    return lax.rsqrt(deg)


def _mm_scale_body(x_ref, w_ref, d0_ref, d1_ref, lo_ref, hi_ref):
    dis = _dis(d0_ref, d1_ref)
    xw = jnp.dot(x_ref[...], w_ref[...], preferred_element_type=jnp.float32)
    xs = xw * dis
    lo_ref[...] = xs[:, :F]
    hi_ref[...] = xs[:, F:]


def _combine_mm_body(alo_ref, ahi_ref, xlo_ref, xhi_ref, d0_ref, d1_ref,
                     b_ref, w_ref, lo_ref, hi_ref):
    dis = _dis(d0_ref, d1_ref)
    tot = jnp.concatenate(
        [alo_ref[...] + xlo_ref[...], ahi_ref[...] + xhi_ref[...]], axis=1)
    h = jnp.maximum(tot * dis + b_ref[...], 0.0)
    xs = jnp.dot(h, w_ref[...], preferred_element_type=jnp.float32) * dis
    lo_ref[...] = xs[:, :F]
    hi_ref[...] = xs[:, F:]


def _final_body(alo_ref, ahi_ref, xlo_ref, xhi_ref, d0_ref, d1_ref, b_ref,
                wl_ref, bl_ref, o_ref):
    dis = _dis(d0_ref, d1_ref)
    tot = jnp.concatenate(
        [alo_ref[...] + xlo_ref[...], ahi_ref[...] + xhi_ref[...]], axis=1)
    h = jnp.maximum(tot * dis + b_ref[...], 0.0)
    o_ref[...] = (
        jnp.dot(h, wl_ref[...], preferred_element_type=jnp.float32) + bl_ref[...]
    )


def _row_spec(cols):
    return pl.BlockSpec((MBLK, cols), lambda i: (i, 0))


def _full_spec(r, cols):
    return pl.BlockSpec((r, cols), lambda i: (0, 0))


_mm_scale = pl.pallas_call(
    _mm_scale_body,
    grid=(GRID,),
    in_specs=[_row_spec(D), _full_spec(D, D), _row_spec(16), _row_spec(16)],
    out_specs=[_row_spec(F), _row_spec(F)],
    out_shape=[jax.ShapeDtypeStruct((N, F), jnp.float32)] * 2,
)

_combine_mm = pl.pallas_call(
    _combine_mm_body,
    grid=(GRID,),
    in_specs=[_row_spec(F), _row_spec(F), _row_spec(F), _row_spec(F),
              _row_spec(16), _row_spec(16), _full_spec(1, D),
              _full_spec(D, D)],
    out_specs=[_row_spec(F), _row_spec(F)],
    out_shape=[jax.ShapeDtypeStruct((N, F), jnp.float32)] * 2,
)

_final_mm = pl.pallas_call(
    _final_body,
    grid=(GRID,),
    in_specs=[_row_spec(F), _row_spec(F), _row_spec(F), _row_spec(F),
              _row_spec(16), _row_spec(16), _full_spec(1, D),
              _full_spec(D, 40), _full_spec(1, 40)],
    out_specs=_row_spec(40),
    out_shape=jax.ShapeDtypeStruct((N, 40), jnp.float32),
)


def kernel(x, edge_index, W1, b1, W2, b2, Wlin, blin):
    ei = edge_index.astype(jnp.int32)
    pad = E_PAD - E
    pad_block = jnp.concatenate(
        [jnp.zeros((1, pad), jnp.int32),
         jnp.full((1, pad), DUMMY_ROW, jnp.int32)], axis=0)
    ei3 = jnp.concatenate([ei, pad_block], axis=1).reshape(2, NCH, CHUNK)

    d0, d1 = _deg_kernel(ei3)

    x1lo, x1hi = _mm_scale(x, W1, d0, d1)
    a1lo, a1hi = _scatter_kernel(x1lo, x1hi, ei3)
    x2lo, x2hi = _combine_mm(a1lo, a1hi, x1lo, x1hi, d0, d1,
                             b1.reshape(1, D), W2)
    a2lo, a2hi = _scatter_kernel(x2lo, x2hi, ei3)
    out = _final_mm(a2lo, a2hi, x2lo, x2hi, d0, d1,
                    b2.reshape(1, D), Wlin.T, blin.reshape(1, 40))
    return out


# speedup vs baseline: 26.1144x; 1.1207x over previous
import functools

import jax
import jax.numpy as jnp
from jax import lax
from jax.experimental import pallas as pl
from jax.experimental.pallas import tpu as pltpu
from jax.experimental.pallas import tpu_sc as plsc

N = 10000
D = 128
F = D // 2
E = 320000
CHUNK = 128
NC = 2
NS = 16
CH_PER_TILE = 160
NCH = NS * CH_PER_TILE
E_PAD = NCH * CHUNK
N_PAD = 10240
ROWS_PER_TILE = N_PAD // NS
TAIL = N - (NS - 1) * ROWS_PER_TILE
DUMMY_ROW = N

_MESH = plsc.VectorSubcoreMesh(core_axis_name="c", subcore_axis_name="s")


def _fill_const(ref, nrows, ncols, val):
    v = jnp.full((16,), val, jnp.float32)

    def body(i, _):
        for k in range(ncols // 16):
            ref[i, pl.ds(k * 16, 16)] = v
        return 0

    lax.fori_loop(0, nrows, body, 0)


def _copy_out(acc_sh, dst_ref, s):
    @pl.when(s < NS - 1)
    def _():
        pltpu.sync_copy(acc_sh.at[pl.ds(s * ROWS_PER_TILE, ROWS_PER_TILE)],
                        dst_ref.at[pl.ds(s * ROWS_PER_TILE, ROWS_PER_TILE)])

    @pl.when(s == NS - 1)
    def _():
        r0 = (NS - 1) * ROWS_PER_TILE
        pltpu.sync_copy(acc_sh.at[pl.ds(r0, TAIL)],
                        dst_ref.at[pl.ds(r0, TAIL)])


DEG_CH_PER_TILE = CH_PER_TILE // 2


@functools.partial(
    pl.kernel,
    out_type=[jax.ShapeDtypeStruct((N, 16), jnp.float32)] * 2,
    mesh=_MESH,
    compiler_params=pltpu.CompilerParams(use_tc_tiling_on_sc=False),
    scratch_types=[
        pltpu.VMEM((CHUNK, 16), jnp.float32),
        pltpu.VMEM((8, CHUNK), jnp.int32),
        pltpu.VMEM((8, CHUNK), jnp.int32),
        pltpu.VMEM((ROWS_PER_TILE, 16), jnp.float32),
        pltpu.VMEM_SHARED((N_PAD, 16), jnp.float32),
        pltpu.SemaphoreType.DMA,
        pltpu.SemaphoreType.DMA,
    ],
)
def _deg_kernel(ei_hbm, out0_hbm, out1_hbm, ones_v, dva, dvb, zero_v, acc_sh,
                sma, smb):
    c = lax.axis_index("c")
    s = lax.axis_index("s")
    dv = [dva, dvb]
    sm = [sma, smb]

    _fill_const(ones_v, CHUNK, 16, 1.0)
    _fill_const(zero_v, ROWS_PER_TILE, 16, 0.0)

    pltpu.sync_copy(zero_v, acc_sh.at[pl.ds(s * ROWS_PER_TILE, ROWS_PER_TILE)])
    plsc.subcore_barrier()

    base = (c * NS + s) * DEG_CH_PER_TILE
    nblk = DEG_CH_PER_TILE // 8

    for m in range(nblk):
        if m >= 2:
            for _ in range(8):
                pltpu.make_async_copy(
                    ones_v, acc_sh.at[pl.ds(0, CHUNK)], sm[m % 2]).wait()
        pltpu.sync_copy(ei_hbm.at[1].at[pl.ds(base + m * 8, 8)], dv[m % 2])
        for k in range(8):
            pltpu.async_copy(
                ones_v, acc_sh.at[dv[m % 2].at[k]], sm[m % 2], add=True)
    for p in range(2):
        for _ in range(8):
            pltpu.make_async_copy(
                ones_v, acc_sh.at[pl.ds(0, CHUNK)], sm[(nblk + p) % 2]).wait()

    plsc.subcore_barrier()

    @pl.when(c == 0)
    def _():
        _copy_out(acc_sh, out0_hbm, s)

    @pl.when(c == 1)
    def _():
        _copy_out(acc_sh, out1_hbm, s)


IBLK = 32
NIB = CH_PER_TILE // IBLK
STAGE = 640


@functools.partial(
    pl.kernel,
    out_type=[jax.ShapeDtypeStruct((N, F), jnp.float32)] * 2,
    mesh=_MESH,
    compiler_params=pltpu.CompilerParams(use_tc_tiling_on_sc=False),
    scratch_types=(
        [pltpu.VMEM((CHUNK, F), jnp.float32)] * 3 +
        [pltpu.VMEM((IBLK, CHUNK), jnp.int32)] * 4 +
        [
            pltpu.VMEM_SHARED((N, F), jnp.float32),
            pltpu.VMEM_SHARED((N_PAD, F), jnp.float32),
        ] +
        [pltpu.SemaphoreType.DMA] * 6
    ),
)
def _scatter_kernel(xs_lo_hbm, xs_hi_hbm, ei_hbm, out0_hbm, out1_hbm,
                    r0, r1, r2, sa, sb, da, db, xs_sp, acc_sh,
                    g0, g1, g2, s0, s1, s2):
    c = lax.axis_index("c")
    s = lax.axis_index("s")
    rows = [r0, r1, r2]
    srcb = [sa, sb]
    dstb = [da, db]
    gsem = [g0, g1, g2]
    ssem = [s0, s1, s2]

    _fill_const(r0, CHUNK, F, 0.0)
    for k in range(ROWS_PER_TILE // CHUNK):
        pltpu.sync_copy(
            r0, acc_sh.at[pl.ds(s * ROWS_PER_TILE + k * CHUNK, CHUNK)])

    def stage(src_half):
        @pl.when(s < NS - 1)
        def _():
            pltpu.sync_copy(src_half.at[pl.ds(s * STAGE, STAGE)],
                            xs_sp.at[pl.ds(s * STAGE, STAGE)])

        @pl.when(s == NS - 1)
        def _():
            pltpu.sync_copy(src_half.at[pl.ds((NS - 1) * STAGE, TAIL)],
                            xs_sp.at[pl.ds((NS - 1) * STAGE, TAIL)])

    @pl.when(c == 0)
    def _():
        stage(xs_lo_hbm)

    @pl.when(c == 1)
    def _():
        stage(xs_hi_hbm)

    plsc.subcore_barrier()

    base = s * CH_PER_TILE

    def load_iblk(m):
        pltpu.sync_copy(
            ei_hbm.at[0].at[pl.ds(base + m * IBLK, IBLK)], srcb[m % 2])
        pltpu.sync_copy(
            ei_hbm.at[1].at[pl.ds(base + m * IBLK, IBLK)], dstb[m % 2])

    def gath(ci, k):
        pltpu.async_copy(
            xs_sp.at[srcb[(ci // IBLK) % 2].at[ci % IBLK]], rows[k], gsem[k])

    def wait_g(k):
        pltpu.make_async_copy(
            xs_sp.at[pl.ds(0, CHUNK)], rows[k], gsem[k]).wait()

    def scat(ci, k):
        pltpu.async_copy(
            rows[k], acc_sh.at[dstb[(ci // IBLK) % 2].at[ci % IBLK]],
            ssem[k], add=True)

    def wait_s(k):
        pltpu.make_async_copy(
            rows[k], acc_sh.at[pl.ds(0, CHUNK)], ssem[k]).wait()

    load_iblk(0)
    gath(0, 0)
    gath(1, 1)
    for m in range(NIB):
        if m > 0:
            wait_s((m * IBLK - 1) % 3)
        if m + 1 < NIB:
            load_iblk(m + 1)
        for k in range(IBLK):
            ci = m * IBLK + k
            sl = ci % 3
            wait_g(sl)
            scat(ci, sl)
            if ci + 2 < CH_PER_TILE:
                if k > 0:
                    wait_s((ci - 1) % 3)
                gath(ci + 2, (ci + 2) % 3)
    for x in (CH_PER_TILE - 3, CH_PER_TILE - 2, CH_PER_TILE - 1):
        wait_s(x % 3)

    plsc.subcore_barrier()

    @pl.when(c == 0)
    def _():
        _copy_out(acc_sh, out0_hbm, s)

    @pl.when(c == 1)
    def _():
        _copy_out(acc_sh, out1_hbm, s)


MBLK = 1000
GRID = N // MBLK


def _dis(d0_ref, d1_ref):
    deg = d0_ref[:, 0:1] + d1_ref[:, 0:1] + 1.0
    return lax.rsqrt(deg)


def _mm_scale_body(x_ref, w_ref, d0_ref, d1_ref, lo_ref, hi_ref):
    dis = _dis(d0_ref, d1_ref)
    xw = jnp.dot(x_ref[...], w_ref[...], preferred_element_type=jnp.float32)
    xs = xw * dis
    lo_ref[...] = xs[:, :F]
    hi_ref[...] = xs[:, F:]


def _combine_mm_body(alo_ref, ahi_ref, xlo_ref, xhi_ref, d0_ref, d1_ref,
                     b_ref, w_ref, lo_ref, hi_ref):
    dis = _dis(d0_ref, d1_ref)
    tot = jnp.concatenate(
        [alo_ref[...] + xlo_ref[...], ahi_ref[...] + xhi_ref[...]], axis=1)
    h = jnp.maximum(tot * dis + b_ref[...], 0.0)
    xs = jnp.dot(h, w_ref[...], preferred_element_type=jnp.float32) * dis
    lo_ref[...] = xs[:, :F]
    hi_ref[...] = xs[:, F:]


def _final_body(alo_ref, ahi_ref, xlo_ref, xhi_ref, d0_ref, d1_ref, b_ref,
                wl_ref, bl_ref, o_ref):
    dis = _dis(d0_ref, d1_ref)
    tot = jnp.concatenate(
        [alo_ref[...] + xlo_ref[...], ahi_ref[...] + xhi_ref[...]], axis=1)
    h = jnp.maximum(tot * dis + b_ref[...], 0.0)
    o_ref[...] = (
        jnp.dot(h, wl_ref[...], preferred_element_type=jnp.float32) + bl_ref[...]
    )


def _row_spec(cols):
    return pl.BlockSpec((MBLK, cols), lambda i: (i, 0))


def _full_spec(r, cols):
    return pl.BlockSpec((r, cols), lambda i: (0, 0))


_mm_scale = pl.pallas_call(
    _mm_scale_body,
    grid=(GRID,),
    in_specs=[_row_spec(D), _full_spec(D, D), _row_spec(16), _row_spec(16)],
    out_specs=[_row_spec(F), _row_spec(F)],
    out_shape=[jax.ShapeDtypeStruct((N, F), jnp.float32)] * 2,
)

_combine_mm = pl.pallas_call(
    _combine_mm_body,
    grid=(GRID,),
    in_specs=[_row_spec(F), _row_spec(F), _row_spec(F), _row_spec(F),
              _row_spec(16), _row_spec(16), _full_spec(1, D),
              _full_spec(D, D)],
    out_specs=[_row_spec(F), _row_spec(F)],
    out_shape=[jax.ShapeDtypeStruct((N, F), jnp.float32)] * 2,
)

_final_mm = pl.pallas_call(
    _final_body,
    grid=(GRID,),
    in_specs=[_row_spec(F), _row_spec(F), _row_spec(F), _row_spec(F),
              _row_spec(16), _row_spec(16), _full_spec(1, D),
              _full_spec(D, 40), _full_spec(1, 40)],
    out_specs=_row_spec(40),
    out_shape=jax.ShapeDtypeStruct((N, 40), jnp.float32),
)


def kernel(x, edge_index, W1, b1, W2, b2, Wlin, blin):
    ei = edge_index.astype(jnp.int32)
    pad = E_PAD - E
    pad_block = jnp.concatenate(
        [jnp.zeros((1, pad), jnp.int32),
         jnp.full((1, pad), DUMMY_ROW, jnp.int32)], axis=0)
    ei3 = jnp.concatenate([ei, pad_block], axis=1).reshape(2, NCH, CHUNK)

    d0, d1 = _deg_kernel(ei3)

    x1lo, x1hi = _mm_scale(x, W1, d0, d1)
    a1lo, a1hi = _scatter_kernel(x1lo, x1hi, ei3)
    x2lo, x2hi = _combine_mm(a1lo, a1hi, x1lo, x1hi, d0, d1,
                             b1.reshape(1, D), W2)
    a2lo, a2hi = _scatter_kernel(x2lo, x2hi, ei3)
    out = _final_mm(a2lo, a2hi, x2lo, x2hi, d0, d1,
                    b2.reshape(1, D), Wlin.T, blin.reshape(1, 40))
    return out
